# baseline (speedup 1.0000x reference)
import functools

import jax
import jax.numpy as jnp
from jax import lax
from jax.experimental import pallas as pl
from jax.experimental.pallas import tpu as pltpu
from jax.experimental.pallas import tpu_sc as plsc

EMBED = 128
ROWS = 17
REP = 4
GROUPS = 50
DIM0 = 16384
NUM_WORKERS = 32
CPW = DIM0 // NUM_WORKERS
CHUNK = 128
QPJ = CPW // CHUNK


def _lookup(idx_hbm, table_hbm, out_hbm, idx_v, table_sh, s0, s1, s2, s3,
            gsem, wsem):
  sid = lax.axis_index("s")
  wid = sid * 2 + lax.axis_index("c")
  col0 = wid * CPW

  def load_table():
    for r in range(REP):
      pltpu.sync_copy(table_hbm, table_sh.at[pl.ds(r * ROWS, ROWS)])

  pl.when(sid == 0)(load_table)
  pltpu.sync_copy(idx_hbm.at[pl.ds(0, GROUPS), pl.ds(col0, CPW)], idx_v)

  rep_off = jnp.broadcast_to((sid % REP) * ROWS, (16,)).astype(jnp.int32)

  def adj(t, _):
    j = t // (CPW // 16)
    k = t % (CPW // 16)
    idx_v[j, pl.ds(k * 16, 16)] = idx_v[j, pl.ds(k * 16, 16)] + rep_off
    return 0

  lax.fori_loop(0, GROUPS * (CPW // 16), adj, 0)
  plsc.subcore_barrier()

  bufs = (s0, s1, s2, s3)

  def start_gather(b, j, q):
    pltpu.async_copy(
        table_sh.at[idx_v.at[j, pl.ds(q * CHUNK, CHUNK)]], bufs[b], gsem)

  def wait_gather(b):
    pltpu.make_async_copy(
        table_sh.at[idx_v.at[0, pl.ds(0, CHUNK)]], bufs[b], gsem).wait()

  def start_write(b, j, q):
    pltpu.async_copy(
        bufs[b], out_hbm.at[j, pl.ds(col0 + q * CHUNK, CHUNK)], wsem)

  def drain_write(b):
    pltpu.make_async_copy(
        bufs[b], out_hbm.at[0, pl.ds(0, CHUNK)], wsem).wait()

  for q in range(3):
    start_gather(q, 0, q)

  def row_body(j, _):
    for q in range(4):
      wait_gather(q)
      start_write(q, j, q)
      if q == 0:
        pl.when(j >= 1)(lambda: drain_write(0))
      else:
        drain_write(q - 1)
      nq = (q + 3) % 4
      nj = j + (q + 3) // 4
      pl.when(nj < GROUPS)(lambda nj=nj, nq=nq: start_gather(nq, nj, nq))
    return 0

  lax.fori_loop(0, GROUPS, row_body, 0)
  drain_write(3)


def kernel(group_idx, table):
  idx_t = jnp.transpose(group_idx)
  mesh = plsc.VectorSubcoreMesh(core_axis_name="c", subcore_axis_name="s")
  run = functools.partial(
      pl.kernel,
      out_type=jax.ShapeDtypeStruct((GROUPS, DIM0, EMBED), jnp.float32),
      mesh=mesh,
      compiler_params=pltpu.CompilerParams(needs_layout_passes=False),
      scratch_types=[
          pltpu.VMEM((GROUPS, CPW), jnp.int32),
          pltpu.VMEM_SHARED((REP * ROWS, EMBED), jnp.float32),
          pltpu.VMEM((CHUNK, EMBED), jnp.float32),
          pltpu.VMEM((CHUNK, EMBED), jnp.float32),
          pltpu.VMEM((CHUNK, EMBED), jnp.float32),
          pltpu.VMEM((CHUNK, EMBED), jnp.float32),
          pltpu.SemaphoreType.DMA,
          pltpu.SemaphoreType.DMA,
      ],
  )(_lookup)
  out_t = run(idx_t, table)
  return jnp.transpose(out_t, (1, 0, 2))
